# baseline (speedup 1.0000x reference)
import functools

import jax
import jax.numpy as jnp
import numpy as np
from jax import lax
from jax.experimental import pallas as pl
from jax.experimental.pallas import tpu as pltpu
from jax.experimental.pallas import tpu_sc as plsc

N_POS = 20000
N_VERT = 5023
N_FACE = 9976
F_PAD = 10240
FACES_PER_WORKER = F_PAD // 32
GROUPS_PER_WORKER = FACES_PER_WORKER // 16
BN = 128
N_PAD = 20096
BF = 128
ONE_THIRD = np.float32(1.0) / np.float32(3.0)
CENT_PAD = np.float32(5e18)
C2_PAD = np.float32(3e38)


def _centroid_sc_kernel(vflat_hbm, facest_hbm, out_hbm, vflat_v, fidx_v, ct_v,
                        sem):
    num_cores = 2
    wid = lax.axis_index("s") * num_cores + lax.axis_index("c")
    base = wid * FACES_PER_WORKER
    pltpu.sync_copy(vflat_hbm, vflat_v)
    for k in range(3):
        pltpu.sync_copy(facest_hbm.at[pl.ds(k * F_PAD + base, FACES_PER_WORKER)],
                        fidx_v.at[pl.ds(k * FACES_PER_WORKER, FACES_PER_WORKER)])
    lane = jnp.arange(16, dtype=jnp.int32)

    def body(g, carry):
        off = g * 16
        i0 = fidx_v[pl.ds(off, 16)] * 3
        i1 = fidx_v[pl.ds(FACES_PER_WORKER + off, 16)] * 3
        i2 = fidx_v[pl.ds(2 * FACES_PER_WORKER + off, 16)] * 3
        valid = (base + off + lane) < N_FACE
        c2 = jnp.zeros((16,), jnp.float32)
        for j in range(3):
            cj = jnp.full((16,), j, jnp.int32)
            v0 = plsc.load_gather(vflat_v, [i0 + cj])
            v1 = plsc.load_gather(vflat_v, [i1 + cj])
            v2 = plsc.load_gather(vflat_v, [i2 + cj])
            cent = ((v0 + v1) + v2) * ONE_THIRD
            cent = jnp.where(valid, cent, CENT_PAD)
            ct_v[pl.ds(j * FACES_PER_WORKER + off, 16)] = cent
            if j == 0:
                c2 = cent * cent
            else:
                c2 = c2 + cent * cent
        ct_v[pl.ds(3 * FACES_PER_WORKER + off, 16)] = jnp.where(valid, c2, C2_PAD)
        return carry

    lax.fori_loop(0, GROUPS_PER_WORKER, body, 0)
    for r in range(4):
        pltpu.sync_copy(ct_v.at[pl.ds(r * FACES_PER_WORKER, FACES_PER_WORKER)],
                        out_hbm.at[pl.ds(r * F_PAD + base, FACES_PER_WORKER)])


def _centroids_sc(vertices_flat, faces_t):
    mesh = plsc.VectorSubcoreMesh(core_axis_name="c", subcore_axis_name="s")
    k = functools.partial(
        pl.kernel,
        out_type=jax.ShapeDtypeStruct((4 * F_PAD,), jnp.float32),
        mesh=mesh,
        scratch_types=[
            pltpu.VMEM(vertices_flat.shape, jnp.float32),
            pltpu.VMEM((3 * FACES_PER_WORKER,), jnp.int32),
            pltpu.VMEM((4 * FACES_PER_WORKER,), jnp.float32),
            pltpu.SemaphoreType.DMA,
        ],
        compiler_params=pltpu.CompilerParams(needs_layout_passes=False),
    )(_centroid_sc_kernel)
    return k(vertices_flat, faces_t).reshape(4, F_PAD)


def _bind_tc_kernel(pos_ref, cb_ref, c2_ref, out_ref):
    p = pos_ref[...]
    x = p[:, 0:1]
    y = p[:, 1:2]
    z = p[:, 2:3]
    p2 = (x * x + y * y) + z * z

    def body(j, carry):
        minv, minj = carry
        cb = cb_ref[:, pl.ds(j * BF, BF)]
        c2 = c2_ref[:, pl.ds(j * BF, BF)]
        dot = lax.dot_general(p, cb, (((1,), (0,)), ((), ())),
                              preferred_element_type=jnp.float32)
        d2 = (p2 + c2) - 2.0 * dot
        better = d2 < minv
        minv = jnp.where(better, d2, minv)
        minj = jnp.where(better, j, minj)
        return minv, minj

    minv0 = jnp.full((BN, BF), jnp.inf, jnp.float32)
    minj0 = jnp.zeros((BN, BF), jnp.int32)
    minv, minj = lax.fori_loop(0, F_PAD // BF, body, (minv0, minj0))

    m = jnp.min(minv, axis=1, keepdims=True)
    lane = lax.broadcasted_iota(jnp.int32, (BN, BF), 1)
    idx = minj * BF + lane
    cand = jnp.where(minv == m, idx, jnp.int32(2**31 - 1))
    out_ref[0, 0, :] = jnp.min(cand, axis=1)


def kernel(positions, vertices, faces):
    faces_t = jnp.pad(faces, ((0, F_PAD - N_FACE), (0, 0))).T.reshape(-1)
    vflat = jnp.pad(vertices.reshape(-1), (0, 3))

    ct = _centroids_sc(vflat, faces_t)
    cb = ct[0:3].astype(jnp.bfloat16)
    c2 = ct[3:4]

    grid = N_PAD // BN
    pos_pad = jnp.pad(positions, ((0, N_PAD - N_POS), (0, 0)))
    out = pl.pallas_call(
        _bind_tc_kernel,
        grid=(grid,),
        in_specs=[
            pl.BlockSpec((BN, 3), lambda i: (i, 0)),
            pl.BlockSpec((3, F_PAD), lambda i: (0, 0)),
            pl.BlockSpec((1, F_PAD), lambda i: (0, 0)),
        ],
        out_specs=pl.BlockSpec((1, 1, BN), lambda i: (i, 0, 0)),
        out_shape=jax.ShapeDtypeStruct((grid, 1, BN), jnp.int32),
    )(pos_pad, cb, c2)
    return out.reshape(N_PAD)[:N_POS]


# speedup vs baseline: 5.9682x; 5.9682x over previous
import functools

import jax
import jax.numpy as jnp
import numpy as np
from jax import lax
from jax.experimental import pallas as pl
from jax.experimental.pallas import tpu as pltpu
from jax.experimental.pallas import tpu_sc as plsc

N_POS = 20000
N_VERT = 5023
N_FACE = 9976
F_PAD = 10240
FACES_PER_WORKER = F_PAD // 32
GROUPS_PER_WORKER = FACES_PER_WORKER // 16
BN = 128
N_PAD = 20096
BF = 128
ONE_THIRD = np.float32(1.0) / np.float32(3.0)
CENT_PAD = np.float32(5e18)
C2_PAD = np.float32(3e38)


def _centroid_sc_kernel(vflat_hbm, facest_hbm, out_hbm, vflat_v, fidx_v, ct_v,
                        sem):
    num_cores = 2
    wid = lax.axis_index("s") * num_cores + lax.axis_index("c")
    base = wid * FACES_PER_WORKER
    pltpu.sync_copy(vflat_hbm, vflat_v)
    for k in range(3):
        pltpu.sync_copy(facest_hbm.at[pl.ds(k * F_PAD + base, FACES_PER_WORKER)],
                        fidx_v.at[pl.ds(k * FACES_PER_WORKER, FACES_PER_WORKER)])
    lane = jnp.arange(16, dtype=jnp.int32)

    def body(g, carry):
        off = g * 16
        i0 = fidx_v[pl.ds(off, 16)] * 3
        i1 = fidx_v[pl.ds(FACES_PER_WORKER + off, 16)] * 3
        i2 = fidx_v[pl.ds(2 * FACES_PER_WORKER + off, 16)] * 3
        valid = (base + off + lane) < N_FACE
        c2 = jnp.zeros((16,), jnp.float32)
        for j in range(3):
            cj = jnp.full((16,), j, jnp.int32)
            v0 = plsc.load_gather(vflat_v, [i0 + cj])
            v1 = plsc.load_gather(vflat_v, [i1 + cj])
            v2 = plsc.load_gather(vflat_v, [i2 + cj])
            cent = ((v0 + v1) + v2) * ONE_THIRD
            cent = jnp.where(valid, cent, CENT_PAD)
            ct_v[pl.ds(j * FACES_PER_WORKER + off, 16)] = cent
            if j == 0:
                c2 = cent * cent
            else:
                c2 = c2 + cent * cent
        ct_v[pl.ds(3 * FACES_PER_WORKER + off, 16)] = jnp.where(valid, c2, C2_PAD)
        return carry

    lax.fori_loop(0, GROUPS_PER_WORKER, body, 0)
    for r in range(4):
        pltpu.sync_copy(ct_v.at[pl.ds(r * FACES_PER_WORKER, FACES_PER_WORKER)],
                        out_hbm.at[pl.ds(r * F_PAD + base, FACES_PER_WORKER)])


def _centroids_sc(vertices_flat, faces_t):
    mesh = plsc.VectorSubcoreMesh(core_axis_name="c", subcore_axis_name="s")
    k = functools.partial(
        pl.kernel,
        out_type=jax.ShapeDtypeStruct((4 * F_PAD,), jnp.float32),
        mesh=mesh,
        scratch_types=[
            pltpu.VMEM(vertices_flat.shape, jnp.float32),
            pltpu.VMEM((3 * FACES_PER_WORKER,), jnp.int32),
            pltpu.VMEM((4 * FACES_PER_WORKER,), jnp.float32),
            pltpu.SemaphoreType.DMA,
        ],
        compiler_params=pltpu.CompilerParams(needs_layout_passes=False),
    )(_centroid_sc_kernel)
    return k(vertices_flat, faces_t).reshape(4, F_PAD)


def _bind_tc_kernel(pos_ref, cb_ref, c2_ref, out_ref):
    p = pos_ref[...]
    x = p[:, 0:1]
    y = p[:, 1:2]
    z = p[:, 2:3]
    p2 = (x * x + y * y) + z * z

    minv = jnp.full((BN, BF), jnp.inf, jnp.float32)
    minj = jnp.zeros((BN, BF), jnp.int32)
    for j in range(F_PAD // BF):
        cb = cb_ref[:, pl.ds(j * BF, BF)]
        c2 = c2_ref[:, pl.ds(j * BF, BF)]
        dot = lax.dot_general(p, cb, (((1,), (0,)), ((), ())),
                              preferred_element_type=jnp.float32)
        d2 = (p2 + c2) - 2.0 * dot
        better = d2 < minv
        minv = jnp.where(better, d2, minv)
        minj = jnp.where(better, j, minj)

    m = jnp.min(minv, axis=1, keepdims=True)
    lane = lax.broadcasted_iota(jnp.int32, (BN, BF), 1)
    idx = minj * BF + lane
    cand = jnp.where(minv == m, idx, jnp.int32(2**31 - 1))
    out_ref[0, 0, :] = jnp.min(cand, axis=1)


def kernel(positions, vertices, faces):
    faces_t = jnp.pad(faces, ((0, F_PAD - N_FACE), (0, 0))).T.reshape(-1)
    vflat = jnp.pad(vertices.reshape(-1), (0, 3))

    ct = _centroids_sc(vflat, faces_t)
    cb = ct[0:3].astype(jnp.bfloat16)
    c2 = ct[3:4]

    grid = N_PAD // BN
    pos_pad = jnp.pad(positions, ((0, N_PAD - N_POS), (0, 0)))
    out = pl.pallas_call(
        _bind_tc_kernel,
        grid=(grid,),
        in_specs=[
            pl.BlockSpec((BN, 3), lambda i: (i, 0)),
            pl.BlockSpec((3, F_PAD), lambda i: (0, 0)),
            pl.BlockSpec((1, F_PAD), lambda i: (0, 0)),
        ],
        out_specs=pl.BlockSpec((1, 1, BN), lambda i: (i, 0, 0)),
        out_shape=jax.ShapeDtypeStruct((grid, 1, BN), jnp.int32),
    )(pos_pad, cb, c2)
    return out.reshape(N_PAD)[:N_POS]


# speedup vs baseline: 6.3831x; 1.0695x over previous
import functools

import jax
import jax.numpy as jnp
import numpy as np
from jax import lax
from jax.experimental import pallas as pl
from jax.experimental.pallas import tpu as pltpu
from jax.experimental.pallas import tpu_sc as plsc

N_POS = 20000
N_VERT = 5023
N_FACE = 9976
F_PAD = 10240
FACES_PER_WORKER = F_PAD // 32
GROUPS_PER_WORKER = FACES_PER_WORKER // 16
BN = 256
N_PAD = 20480
BF = 128
ONE_THIRD = np.float32(1.0) / np.float32(3.0)
CENT_PAD = np.float32(5e18)
C2_PAD = np.float32(3e38)


def _centroid_sc_kernel(vflat_hbm, facest_hbm, out_hbm, vflat_v, fidx_v, ct_v,
                        sem):
    num_cores = 2
    wid = lax.axis_index("s") * num_cores + lax.axis_index("c")
    base = wid * FACES_PER_WORKER
    pltpu.sync_copy(vflat_hbm, vflat_v)
    for k in range(3):
        pltpu.sync_copy(facest_hbm.at[pl.ds(k * F_PAD + base, FACES_PER_WORKER)],
                        fidx_v.at[pl.ds(k * FACES_PER_WORKER, FACES_PER_WORKER)])
    lane = jnp.arange(16, dtype=jnp.int32)

    def body(g, carry):
        off = g * 16
        i0 = fidx_v[pl.ds(off, 16)] * 3
        i1 = fidx_v[pl.ds(FACES_PER_WORKER + off, 16)] * 3
        i2 = fidx_v[pl.ds(2 * FACES_PER_WORKER + off, 16)] * 3
        valid = (base + off + lane) < N_FACE
        c2 = jnp.zeros((16,), jnp.float32)
        for j in range(3):
            cj = jnp.full((16,), j, jnp.int32)
            v0 = plsc.load_gather(vflat_v, [i0 + cj])
            v1 = plsc.load_gather(vflat_v, [i1 + cj])
            v2 = plsc.load_gather(vflat_v, [i2 + cj])
            cent = ((v0 + v1) + v2) * ONE_THIRD
            cent = jnp.where(valid, cent, CENT_PAD)
            ct_v[pl.ds(j * FACES_PER_WORKER + off, 16)] = cent
            if j == 0:
                c2 = cent * cent
            else:
                c2 = c2 + cent * cent
        ct_v[pl.ds(3 * FACES_PER_WORKER + off, 16)] = jnp.where(valid, c2, C2_PAD)
        return carry

    lax.fori_loop(0, GROUPS_PER_WORKER, body, 0)
    for r in range(4):
        pltpu.sync_copy(ct_v.at[pl.ds(r * FACES_PER_WORKER, FACES_PER_WORKER)],
                        out_hbm.at[pl.ds(r * F_PAD + base, FACES_PER_WORKER)])


def _centroids_sc(vertices_flat, faces_t):
    mesh = plsc.VectorSubcoreMesh(core_axis_name="c", subcore_axis_name="s")
    k = functools.partial(
        pl.kernel,
        out_type=jax.ShapeDtypeStruct((4 * F_PAD,), jnp.float32),
        mesh=mesh,
        scratch_types=[
            pltpu.VMEM(vertices_flat.shape, jnp.float32),
            pltpu.VMEM((3 * FACES_PER_WORKER,), jnp.int32),
            pltpu.VMEM((4 * FACES_PER_WORKER,), jnp.float32),
            pltpu.SemaphoreType.DMA,
        ],
        compiler_params=pltpu.CompilerParams(needs_layout_passes=False),
    )(_centroid_sc_kernel)
    return k(vertices_flat, faces_t).reshape(4, F_PAD)


def _bind_tc_kernel(pos_ref, cb_ref, c2_ref, out_ref):
    p = pos_ref[...]
    x = p[:, 0:1]
    y = p[:, 1:2]
    z = p[:, 2:3]
    p2 = (x * x + y * y) + z * z

    minv = jnp.full((BN, BF), jnp.inf, jnp.float32)
    minj = jnp.zeros((BN, BF), jnp.int32)
    for j in range(F_PAD // BF):
        cb = cb_ref[:, pl.ds(j * BF, BF)]
        c2 = c2_ref[:, pl.ds(j * BF, BF)]
        dot = lax.dot_general(p, cb, (((1,), (0,)), ((), ())),
                              preferred_element_type=jnp.float32)
        d2 = (p2 + c2) - 2.0 * dot
        better = d2 < minv
        minv = jnp.where(better, d2, minv)
        minj = jnp.where(better, j, minj)

    m = jnp.min(minv, axis=1, keepdims=True)
    lane = lax.broadcasted_iota(jnp.int32, (BN, BF), 1)
    idx = minj * BF + lane
    cand = jnp.where(minv == m, idx, jnp.int32(2**31 - 1))
    out_ref[0, 0, :] = jnp.min(cand, axis=1)


def kernel(positions, vertices, faces):
    faces_t = jnp.pad(faces, ((0, F_PAD - N_FACE), (0, 0))).T.reshape(-1)
    vflat = jnp.pad(vertices.reshape(-1), (0, 3))

    ct = _centroids_sc(vflat, faces_t)
    cb = ct[0:3].astype(jnp.bfloat16)
    c2 = ct[3:4]

    grid = N_PAD // BN
    pos_pad = jnp.pad(positions, ((0, N_PAD - N_POS), (0, 0)))
    out = pl.pallas_call(
        _bind_tc_kernel,
        grid=(grid,),
        in_specs=[
            pl.BlockSpec((BN, 3), lambda i: (i, 0)),
            pl.BlockSpec((3, F_PAD), lambda i: (0, 0)),
            pl.BlockSpec((1, F_PAD), lambda i: (0, 0)),
        ],
        out_specs=pl.BlockSpec((1, 1, BN), lambda i: (i, 0, 0)),
        out_shape=jax.ShapeDtypeStruct((grid, 1, BN), jnp.int32),
    )(pos_pad, cb, c2)
    return out.reshape(N_PAD)[:N_POS]


# speedup vs baseline: 6.6630x; 1.0439x over previous
import functools

import jax
import jax.numpy as jnp
import numpy as np
from jax import lax
from jax.experimental import pallas as pl
from jax.experimental.pallas import tpu as pltpu
from jax.experimental.pallas import tpu_sc as plsc

N_POS = 20000
N_VERT = 5023
N_FACE = 9976
F_PAD = 10240
FACES_PER_WORKER = F_PAD // 32
GROUPS_PER_WORKER = FACES_PER_WORKER // 16
BN = 256
N_PAD = 20480
BF = 128
F_BLOCKS = 78
ONE_THIRD = np.float32(1.0) / np.float32(3.0)
CENT_PAD = np.float32(5e18)
C2_PAD = np.float32(3e38)


def _centroid_sc_kernel(vflat_hbm, facest_hbm, out_hbm, vflat_v, fidx_v, ct_v,
                        sem):
    num_cores = 2
    wid = lax.axis_index("s") * num_cores + lax.axis_index("c")
    base = wid * FACES_PER_WORKER
    pltpu.sync_copy(vflat_hbm, vflat_v)
    for k in range(3):
        pltpu.sync_copy(facest_hbm.at[pl.ds(k * F_PAD + base, FACES_PER_WORKER)],
                        fidx_v.at[pl.ds(k * FACES_PER_WORKER, FACES_PER_WORKER)])
    lane = jnp.arange(16, dtype=jnp.int32)

    def body(g, carry):
        off = g * 16
        i0 = fidx_v[pl.ds(off, 16)] * 3
        i1 = fidx_v[pl.ds(FACES_PER_WORKER + off, 16)] * 3
        i2 = fidx_v[pl.ds(2 * FACES_PER_WORKER + off, 16)] * 3
        valid = (base + off + lane) < N_FACE
        c2 = jnp.zeros((16,), jnp.float32)
        for j in range(3):
            cj = jnp.full((16,), j, jnp.int32)
            v0 = plsc.load_gather(vflat_v, [i0 + cj])
            v1 = plsc.load_gather(vflat_v, [i1 + cj])
            v2 = plsc.load_gather(vflat_v, [i2 + cj])
            cent = ((v0 + v1) + v2) * ONE_THIRD
            cent = jnp.where(valid, cent, CENT_PAD)
            ct_v[pl.ds(j * FACES_PER_WORKER + off, 16)] = cent
            if j == 0:
                c2 = cent * cent
            else:
                c2 = c2 + cent * cent
        ct_v[pl.ds(3 * FACES_PER_WORKER + off, 16)] = jnp.where(valid, c2, C2_PAD)
        return carry

    lax.fori_loop(0, GROUPS_PER_WORKER, body, 0)
    for r in range(4):
        pltpu.sync_copy(ct_v.at[pl.ds(r * FACES_PER_WORKER, FACES_PER_WORKER)],
                        out_hbm.at[pl.ds(r * F_PAD + base, FACES_PER_WORKER)])


def _centroids_sc(vertices_flat, faces_t):
    mesh = plsc.VectorSubcoreMesh(core_axis_name="c", subcore_axis_name="s")
    k = functools.partial(
        pl.kernel,
        out_type=jax.ShapeDtypeStruct((4 * F_PAD,), jnp.float32),
        mesh=mesh,
        scratch_types=[
            pltpu.VMEM(vertices_flat.shape, jnp.float32),
            pltpu.VMEM((3 * FACES_PER_WORKER,), jnp.int32),
            pltpu.VMEM((4 * FACES_PER_WORKER,), jnp.float32),
            pltpu.SemaphoreType.DMA,
        ],
        compiler_params=pltpu.CompilerParams(needs_layout_passes=False),
    )(_centroid_sc_kernel)
    return k(vertices_flat, faces_t).reshape(4, F_PAD)


def _bind_tc_kernel(pos_ref, cb_ref, c2_ref, out_ref):
    p = pos_ref[...]
    x = p[:, 0:1]
    y = p[:, 1:2]
    z = p[:, 2:3]
    p2 = (x * x + y * y) + z * z

    lane_f = lax.broadcasted_iota(jnp.int32, (1, BF), 1).astype(jnp.float32)
    minv = jnp.full((BN, BF), jnp.inf, jnp.float32)
    minjf = jnp.zeros((BN, BF), jnp.float32)
    for j in range(F_BLOCKS):
        cb = cb_ref[:, pl.ds(j * BF, BF)]
        c2 = c2_ref[:, pl.ds(j * BF, BF)]
        dot = lax.dot_general(p, cb, (((1,), (0,)), ((), ())),
                              preferred_element_type=jnp.float32)
        d2 = (p2 + c2) - 2.0 * dot
        better = d2 < minv
        minv = jnp.where(better, d2, minv)
        minjf = jnp.where(better, lane_f + np.float32(j * BF), minjf)

    m = jnp.min(minv, axis=1, keepdims=True)
    cand = jnp.where(minv == m, minjf, np.float32(3e38))
    out_ref[0, 0, :] = jnp.min(cand, axis=1).astype(jnp.int32)


def kernel(positions, vertices, faces):
    faces_t = jnp.pad(faces, ((0, F_PAD - N_FACE), (0, 0))).T.reshape(-1)
    vflat = jnp.pad(vertices.reshape(-1), (0, 3))

    ct = _centroids_sc(vflat, faces_t)
    cb = ct[0:3].astype(jnp.bfloat16)
    c2 = ct[3:4]

    grid = N_PAD // BN
    pos_pad = jnp.pad(positions, ((0, N_PAD - N_POS), (0, 0)))
    out = pl.pallas_call(
        _bind_tc_kernel,
        grid=(grid,),
        in_specs=[
            pl.BlockSpec((BN, 3), lambda i: (i, 0)),
            pl.BlockSpec((3, F_PAD), lambda i: (0, 0)),
            pl.BlockSpec((1, F_PAD), lambda i: (0, 0)),
        ],
        out_specs=pl.BlockSpec((1, 1, BN), lambda i: (i, 0, 0)),
        out_shape=jax.ShapeDtypeStruct((grid, 1, BN), jnp.int32),
    )(pos_pad, cb, c2)
    return out.reshape(N_PAD)[:N_POS]


# speedup vs baseline: 6.7958x; 1.0199x over previous
import functools

import jax
import jax.numpy as jnp
import numpy as np
from jax import lax
from jax.experimental import pallas as pl
from jax.experimental.pallas import tpu as pltpu
from jax.experimental.pallas import tpu_sc as plsc

N_POS = 20000
N_VERT = 5023
N_FACE = 9976
F_PAD = 10240
FACES_PER_WORKER = F_PAD // 32
GROUPS_PER_WORKER = FACES_PER_WORKER // 16
BN = 512
N_PAD = 20480
BF = 128
F_BLOCKS = 78
ONE_THIRD = np.float32(1.0) / np.float32(3.0)
CENT_PAD = np.float32(5e18)
C2_PAD = np.float32(3e38)


def _centroid_sc_kernel(vflat_hbm, facest_hbm, out_hbm, vflat_v, fidx_v, ct_v,
                        sem):
    num_cores = 2
    wid = lax.axis_index("s") * num_cores + lax.axis_index("c")
    base = wid * FACES_PER_WORKER
    pltpu.sync_copy(vflat_hbm, vflat_v)
    for k in range(3):
        pltpu.sync_copy(facest_hbm.at[pl.ds(k * F_PAD + base, FACES_PER_WORKER)],
                        fidx_v.at[pl.ds(k * FACES_PER_WORKER, FACES_PER_WORKER)])
    lane = jnp.arange(16, dtype=jnp.int32)

    def body(g, carry):
        off = g * 16
        i0 = fidx_v[pl.ds(off, 16)] * 3
        i1 = fidx_v[pl.ds(FACES_PER_WORKER + off, 16)] * 3
        i2 = fidx_v[pl.ds(2 * FACES_PER_WORKER + off, 16)] * 3
        valid = (base + off + lane) < N_FACE
        c2 = jnp.zeros((16,), jnp.float32)
        for j in range(3):
            cj = jnp.full((16,), j, jnp.int32)
            v0 = plsc.load_gather(vflat_v, [i0 + cj])
            v1 = plsc.load_gather(vflat_v, [i1 + cj])
            v2 = plsc.load_gather(vflat_v, [i2 + cj])
            cent = ((v0 + v1) + v2) * ONE_THIRD
            cent = jnp.where(valid, cent, CENT_PAD)
            ct_v[pl.ds(j * FACES_PER_WORKER + off, 16)] = cent
            if j == 0:
                c2 = cent * cent
            else:
                c2 = c2 + cent * cent
        ct_v[pl.ds(3 * FACES_PER_WORKER + off, 16)] = jnp.where(valid, c2, C2_PAD)
        return carry

    lax.fori_loop(0, GROUPS_PER_WORKER, body, 0)
    for r in range(4):
        pltpu.sync_copy(ct_v.at[pl.ds(r * FACES_PER_WORKER, FACES_PER_WORKER)],
                        out_hbm.at[pl.ds(r * F_PAD + base, FACES_PER_WORKER)])


def _centroids_sc(vertices_flat, faces_t):
    mesh = plsc.VectorSubcoreMesh(core_axis_name="c", subcore_axis_name="s")
    k = functools.partial(
        pl.kernel,
        out_type=jax.ShapeDtypeStruct((4 * F_PAD,), jnp.float32),
        mesh=mesh,
        scratch_types=[
            pltpu.VMEM(vertices_flat.shape, jnp.float32),
            pltpu.VMEM((3 * FACES_PER_WORKER,), jnp.int32),
            pltpu.VMEM((4 * FACES_PER_WORKER,), jnp.float32),
            pltpu.SemaphoreType.DMA,
        ],
        compiler_params=pltpu.CompilerParams(needs_layout_passes=False),
    )(_centroid_sc_kernel)
    return k(vertices_flat, faces_t).reshape(4, F_PAD)


def _bind_tc_kernel(pos_ref, cb_ref, c2_ref, out_ref):
    p = pos_ref[...]
    x = p[:, 0:1]
    y = p[:, 1:2]
    z = p[:, 2:3]
    p2 = (x * x + y * y) + z * z

    lane_f = lax.broadcasted_iota(jnp.int32, (1, BF), 1).astype(jnp.float32)
    minv = jnp.full((BN, BF), jnp.inf, jnp.float32)
    minjf = jnp.zeros((BN, BF), jnp.float32)
    for j in range(F_BLOCKS):
        cb = cb_ref[:, pl.ds(j * BF, BF)]
        c2 = c2_ref[:, pl.ds(j * BF, BF)]
        dot = lax.dot_general(p, cb, (((1,), (0,)), ((), ())),
                              preferred_element_type=jnp.float32)
        d2 = (p2 + c2) - 2.0 * dot
        better = d2 < minv
        minv = jnp.where(better, d2, minv)
        minjf = jnp.where(better, lane_f + np.float32(j * BF), minjf)

    m = jnp.min(minv, axis=1, keepdims=True)
    cand = jnp.where(minv == m, minjf, np.float32(3e38))
    out_ref[0, 0, :] = jnp.min(cand, axis=1).astype(jnp.int32)


def kernel(positions, vertices, faces):
    faces_t = jnp.pad(faces, ((0, F_PAD - N_FACE), (0, 0))).T.reshape(-1)
    vflat = jnp.pad(vertices.reshape(-1), (0, 3))

    ct = _centroids_sc(vflat, faces_t)
    cb = ct[0:3].astype(jnp.bfloat16)
    c2 = ct[3:4]

    grid = N_PAD // BN
    pos_pad = jnp.pad(positions, ((0, N_PAD - N_POS), (0, 0)))
    out = pl.pallas_call(
        _bind_tc_kernel,
        grid=(grid,),
        in_specs=[
            pl.BlockSpec((BN, 3), lambda i: (i, 0)),
            pl.BlockSpec((3, F_PAD), lambda i: (0, 0)),
            pl.BlockSpec((1, F_PAD), lambda i: (0, 0)),
        ],
        out_specs=pl.BlockSpec((1, 1, BN), lambda i: (i, 0, 0)),
        out_shape=jax.ShapeDtypeStruct((grid, 1, BN), jnp.int32),
    )(pos_pad, cb, c2)
    return out.reshape(N_PAD)[:N_POS]


# speedup vs baseline: 7.4556x; 1.0971x over previous
import functools

import jax
import jax.numpy as jnp
import numpy as np
from jax import lax
from jax.experimental import pallas as pl
from jax.experimental.pallas import tpu as pltpu
from jax.experimental.pallas import tpu_sc as plsc

N_POS = 20000
N_VERT = 5023
N_FACE = 9976
F_PAD = 10240
FACES_PER_WORKER = F_PAD // 32
GROUPS_PER_WORKER = FACES_PER_WORKER // 16
BN = 512
N_PAD = 20480
BF = 128
F_BLOCKS = 78
ONE_THIRD = np.float32(1.0) / np.float32(3.0)
CENT_PAD = np.float32(5e18)
C2_PAD = np.float32(3e38)


def _centroid_sc_kernel(vflat_hbm, facest_hbm, out_hbm, vflat_v, fidx_v, ct_v,
                        sem):
    num_cores = 2
    wid = lax.axis_index("s") * num_cores + lax.axis_index("c")
    base = wid * FACES_PER_WORKER
    pltpu.sync_copy(vflat_hbm, vflat_v)
    for k in range(3):
        pltpu.sync_copy(facest_hbm.at[pl.ds(k * F_PAD + base, FACES_PER_WORKER)],
                        fidx_v.at[pl.ds(k * FACES_PER_WORKER, FACES_PER_WORKER)])
    lane = jnp.arange(16, dtype=jnp.int32)

    def body(g, carry):
        off = g * 16
        i0 = fidx_v[pl.ds(off, 16)] * 3
        i1 = fidx_v[pl.ds(FACES_PER_WORKER + off, 16)] * 3
        i2 = fidx_v[pl.ds(2 * FACES_PER_WORKER + off, 16)] * 3
        valid = (base + off + lane) < N_FACE
        c2 = jnp.zeros((16,), jnp.float32)
        for j in range(3):
            cj = jnp.full((16,), j, jnp.int32)
            v0 = plsc.load_gather(vflat_v, [i0 + cj])
            v1 = plsc.load_gather(vflat_v, [i1 + cj])
            v2 = plsc.load_gather(vflat_v, [i2 + cj])
            cent = ((v0 + v1) + v2) * ONE_THIRD
            cent = jnp.where(valid, cent, CENT_PAD)
            ct_v[pl.ds(j * FACES_PER_WORKER + off, 16)] = cent
            if j == 0:
                c2 = cent * cent
            else:
                c2 = c2 + cent * cent
        ct_v[pl.ds(3 * FACES_PER_WORKER + off, 16)] = jnp.where(valid, c2, C2_PAD)
        return carry

    lax.fori_loop(0, GROUPS_PER_WORKER, body, 0)
    for r in range(4):
        pltpu.sync_copy(ct_v.at[pl.ds(r * FACES_PER_WORKER, FACES_PER_WORKER)],
                        out_hbm.at[pl.ds(r * F_PAD + base, FACES_PER_WORKER)])


def _centroids_sc(vertices_flat, faces_t):
    mesh = plsc.VectorSubcoreMesh(core_axis_name="c", subcore_axis_name="s")
    k = functools.partial(
        pl.kernel,
        out_type=jax.ShapeDtypeStruct((4 * F_PAD,), jnp.float32),
        mesh=mesh,
        scratch_types=[
            pltpu.VMEM(vertices_flat.shape, jnp.float32),
            pltpu.VMEM((3 * FACES_PER_WORKER,), jnp.int32),
            pltpu.VMEM((4 * FACES_PER_WORKER,), jnp.float32),
            pltpu.SemaphoreType.DMA,
        ],
        compiler_params=pltpu.CompilerParams(needs_layout_passes=False),
    )(_centroid_sc_kernel)
    return k(vertices_flat, faces_t).reshape(4, F_PAD)


def _bind_tc_kernel(pos_ref, cb_ref, c2_ref, out_ref):
    p = pos_ref[...]
    x = p[:, 0:1]
    y = p[:, 1:2]
    z = p[:, 2:3]
    p2 = (x * x + y * y) + z * z

    lane_f = lax.broadcasted_iota(jnp.int32, (1, BF), 1).astype(jnp.float32)
    minv = jnp.full((BN, BF), jnp.inf, jnp.float32)
    minjf = jnp.zeros((BN, BF), jnp.float32)
    for j in range(F_BLOCKS):
        cb = cb_ref[:, pl.ds(j * BF, BF)]
        c2 = c2_ref[:, pl.ds(j * BF, BF)]
        dot2 = lax.dot_general(p, cb, (((1,), (0,)), ((), ())),
                               preferred_element_type=jnp.float32)
        d2 = (p2 + c2) - dot2
        better = d2 < minv
        minv = jnp.where(better, d2, minv)
        minjf = jnp.where(better, lane_f + np.float32(j * BF), minjf)

    m = jnp.min(minv, axis=1, keepdims=True)
    cand = jnp.where(minv == m, minjf, np.float32(3e38))
    out_ref[0, 0, :] = jnp.min(cand, axis=1).astype(jnp.int32)


def kernel(positions, vertices, faces):
    faces_t = jnp.pad(faces, ((0, F_PAD - N_FACE), (0, 0))).T.reshape(-1)
    vflat = jnp.pad(vertices.reshape(-1), (0, 3))

    ct = _centroids_sc(vflat, faces_t)
    cb = ct[0:3].astype(jnp.bfloat16) * jnp.bfloat16(2.0)
    c2 = ct[3:4]

    grid = N_PAD // BN
    pos_pad = jnp.pad(positions, ((0, N_PAD - N_POS), (0, 0)))
    out = pl.pallas_call(
        _bind_tc_kernel,
        grid=(grid,),
        in_specs=[
            pl.BlockSpec((BN, 3), lambda i: (i, 0)),
            pl.BlockSpec((3, F_PAD), lambda i: (0, 0)),
            pl.BlockSpec((1, F_PAD), lambda i: (0, 0)),
        ],
        out_specs=pl.BlockSpec((1, 1, BN), lambda i: (i, 0, 0)),
        out_shape=jax.ShapeDtypeStruct((grid, 1, BN), jnp.int32),
    )(pos_pad, cb, c2)
    return out.reshape(N_PAD)[:N_POS]


# speedup vs baseline: 7.5492x; 1.0125x over previous
import functools

import jax
import jax.numpy as jnp
import numpy as np
from jax import lax
from jax.experimental import pallas as pl
from jax.experimental.pallas import tpu as pltpu
from jax.experimental.pallas import tpu_sc as plsc

N_POS = 20000
N_VERT = 5023
N_FACE = 9976
F_PAD = 10240
FACES_PER_WORKER = F_PAD // 32
GROUPS_PER_WORKER = FACES_PER_WORKER // 16
BN = 512
N_PAD = 20480
BF = 128
F_BLOCKS = 78
RC = 64
ONE_THIRD = np.float32(1.0) / np.float32(3.0)
CENT_PAD = np.float32(5e18)
C2_PAD = np.float32(3e38)


def _centroid_sc_kernel(vflat_hbm, facest_hbm, out_hbm, vflat_v, fidx_v, ct_v,
                        sem):
    num_cores = 2
    wid = lax.axis_index("s") * num_cores + lax.axis_index("c")
    base = wid * FACES_PER_WORKER
    pltpu.sync_copy(vflat_hbm, vflat_v)
    for k in range(3):
        pltpu.sync_copy(facest_hbm.at[pl.ds(k * F_PAD + base, FACES_PER_WORKER)],
                        fidx_v.at[pl.ds(k * FACES_PER_WORKER, FACES_PER_WORKER)])
    lane = jnp.arange(16, dtype=jnp.int32)

    def body(g, carry):
        off = g * 16
        i0 = fidx_v[pl.ds(off, 16)] * 3
        i1 = fidx_v[pl.ds(FACES_PER_WORKER + off, 16)] * 3
        i2 = fidx_v[pl.ds(2 * FACES_PER_WORKER + off, 16)] * 3
        valid = (base + off + lane) < N_FACE
        c2 = jnp.zeros((16,), jnp.float32)
        for j in range(3):
            cj = jnp.full((16,), j, jnp.int32)
            v0 = plsc.load_gather(vflat_v, [i0 + cj])
            v1 = plsc.load_gather(vflat_v, [i1 + cj])
            v2 = plsc.load_gather(vflat_v, [i2 + cj])
            cent = ((v0 + v1) + v2) * ONE_THIRD
            cent = jnp.where(valid, cent, CENT_PAD)
            ct_v[pl.ds(j * FACES_PER_WORKER + off, 16)] = cent
            if j == 0:
                c2 = cent * cent
            else:
                c2 = c2 + cent * cent
        ct_v[pl.ds(3 * FACES_PER_WORKER + off, 16)] = jnp.where(valid, c2, C2_PAD)
        return carry

    lax.fori_loop(0, GROUPS_PER_WORKER, body, 0)
    for r in range(4):
        pltpu.sync_copy(ct_v.at[pl.ds(r * FACES_PER_WORKER, FACES_PER_WORKER)],
                        out_hbm.at[pl.ds(r * F_PAD + base, FACES_PER_WORKER)])


def _centroids_sc(vertices_flat, faces_t):
    mesh = plsc.VectorSubcoreMesh(core_axis_name="c", subcore_axis_name="s")
    k = functools.partial(
        pl.kernel,
        out_type=jax.ShapeDtypeStruct((4 * F_PAD,), jnp.float32),
        mesh=mesh,
        scratch_types=[
            pltpu.VMEM(vertices_flat.shape, jnp.float32),
            pltpu.VMEM((3 * FACES_PER_WORKER,), jnp.int32),
            pltpu.VMEM((4 * FACES_PER_WORKER,), jnp.float32),
            pltpu.SemaphoreType.DMA,
        ],
        compiler_params=pltpu.CompilerParams(needs_layout_passes=False),
    )(_centroid_sc_kernel)
    return k(vertices_flat, faces_t).reshape(4, F_PAD)


def _bind_tc_kernel(pos_ref, cb_ref, c2_ref, out_ref, d2_ref):
    p = pos_ref[...]
    x = p[:, 0:1]
    y = p[:, 1:2]
    z = p[:, 2:3]
    p2 = (x * x + y * y) + z * z

    for j in range(F_BLOCKS):
        cb = cb_ref[:, pl.ds(j * BF, BF)]
        c2 = c2_ref[:, pl.ds(j * BF, BF)]
        dot2 = lax.dot_general(p, cb, (((1,), (0,)), ((), ())),
                               preferred_element_type=jnp.float32)
        d2_ref[:, pl.ds(j * BF, BF)] = (p2 + c2) - dot2

    lane_f = lax.broadcasted_iota(jnp.int32, (1, BF), 1).astype(jnp.float32)
    for rc in range(BN // RC):
        minv = jnp.full((RC, BF), jnp.inf, jnp.float32)
        minjf = jnp.zeros((RC, BF), jnp.float32)
        for j in range(F_BLOCKS):
            d2 = d2_ref[pl.ds(rc * RC, RC), pl.ds(j * BF, BF)]
            better = d2 < minv
            minv = jnp.where(better, d2, minv)
            minjf = jnp.where(better, lane_f + np.float32(j * BF), minjf)
        m = jnp.min(minv, axis=1, keepdims=True)
        cand = jnp.where(minv == m, minjf, np.float32(3e38))
        out_ref[0, 0, pl.ds(rc * RC, RC)] = jnp.min(cand, axis=1).astype(jnp.int32)


def kernel(positions, vertices, faces):
    faces_t = jnp.pad(faces, ((0, F_PAD - N_FACE), (0, 0))).T.reshape(-1)
    vflat = jnp.pad(vertices.reshape(-1), (0, 3))

    ct = _centroids_sc(vflat, faces_t)
    cb = ct[0:3].astype(jnp.bfloat16) * jnp.bfloat16(2.0)
    c2 = ct[3:4]

    grid = N_PAD // BN
    pos_pad = jnp.pad(positions, ((0, N_PAD - N_POS), (0, 0)))
    out = pl.pallas_call(
        _bind_tc_kernel,
        grid=(grid,),
        in_specs=[
            pl.BlockSpec((BN, 3), lambda i: (i, 0)),
            pl.BlockSpec((3, F_PAD), lambda i: (0, 0)),
            pl.BlockSpec((1, F_PAD), lambda i: (0, 0)),
        ],
        out_specs=pl.BlockSpec((1, 1, BN), lambda i: (i, 0, 0)),
        out_shape=jax.ShapeDtypeStruct((grid, 1, BN), jnp.int32),
        scratch_shapes=[pltpu.VMEM((BN, F_BLOCKS * BF), jnp.float32)],
    )(pos_pad, cb, c2)
    return out.reshape(N_PAD)[:N_POS]


# speedup vs baseline: 7.6460x; 1.0128x over previous
import functools

import jax
import jax.numpy as jnp
import numpy as np
from jax import lax
from jax.experimental import pallas as pl
from jax.experimental.pallas import tpu as pltpu
from jax.experimental.pallas import tpu_sc as plsc

N_POS = 20000
N_VERT = 5023
N_FACE = 9976
F_PAD = 10240
FACES_PER_WORKER = F_PAD // 32
GROUPS_PER_WORKER = FACES_PER_WORKER // 16
BN = 512
N_PAD = 20480
BF = 256
F_BLOCKS = 39
RC = 64
ONE_THIRD = np.float32(1.0) / np.float32(3.0)
CENT_PAD = np.float32(5e18)
C2_PAD = np.float32(3e38)


def _centroid_sc_kernel(vflat_hbm, facest_hbm, out_hbm, vflat_v, fidx_v, ct_v,
                        sem):
    num_cores = 2
    wid = lax.axis_index("s") * num_cores + lax.axis_index("c")
    base = wid * FACES_PER_WORKER
    pltpu.sync_copy(vflat_hbm, vflat_v)
    for k in range(3):
        pltpu.sync_copy(facest_hbm.at[pl.ds(k * F_PAD + base, FACES_PER_WORKER)],
                        fidx_v.at[pl.ds(k * FACES_PER_WORKER, FACES_PER_WORKER)])
    lane = jnp.arange(16, dtype=jnp.int32)

    def body(g, carry):
        off = g * 16
        i0 = fidx_v[pl.ds(off, 16)] * 3
        i1 = fidx_v[pl.ds(FACES_PER_WORKER + off, 16)] * 3
        i2 = fidx_v[pl.ds(2 * FACES_PER_WORKER + off, 16)] * 3
        valid = (base + off + lane) < N_FACE
        c2 = jnp.zeros((16,), jnp.float32)
        for j in range(3):
            cj = jnp.full((16,), j, jnp.int32)
            v0 = plsc.load_gather(vflat_v, [i0 + cj])
            v1 = plsc.load_gather(vflat_v, [i1 + cj])
            v2 = plsc.load_gather(vflat_v, [i2 + cj])
            cent = ((v0 + v1) + v2) * ONE_THIRD
            cent = jnp.where(valid, cent, CENT_PAD)
            ct_v[pl.ds(j * FACES_PER_WORKER + off, 16)] = cent
            if j == 0:
                c2 = cent * cent
            else:
                c2 = c2 + cent * cent
        ct_v[pl.ds(3 * FACES_PER_WORKER + off, 16)] = jnp.where(valid, c2, C2_PAD)
        return carry

    lax.fori_loop(0, GROUPS_PER_WORKER, body, 0)
    for r in range(4):
        pltpu.sync_copy(ct_v.at[pl.ds(r * FACES_PER_WORKER, FACES_PER_WORKER)],
                        out_hbm.at[pl.ds(r * F_PAD + base, FACES_PER_WORKER)])


def _centroids_sc(vertices_flat, faces_t):
    mesh = plsc.VectorSubcoreMesh(core_axis_name="c", subcore_axis_name="s")
    k = functools.partial(
        pl.kernel,
        out_type=jax.ShapeDtypeStruct((4 * F_PAD,), jnp.float32),
        mesh=mesh,
        scratch_types=[
            pltpu.VMEM(vertices_flat.shape, jnp.float32),
            pltpu.VMEM((3 * FACES_PER_WORKER,), jnp.int32),
            pltpu.VMEM((4 * FACES_PER_WORKER,), jnp.float32),
            pltpu.SemaphoreType.DMA,
        ],
        compiler_params=pltpu.CompilerParams(needs_layout_passes=False),
    )(_centroid_sc_kernel)
    return k(vertices_flat, faces_t).reshape(4, F_PAD)


def _bind_tc_kernel(pos_ref, cb_ref, c2_ref, out_ref, d2_ref):
    p = pos_ref[...]
    x = p[:, 0:1]
    y = p[:, 1:2]
    z = p[:, 2:3]
    p2 = (x * x + y * y) + z * z

    for j in range(F_BLOCKS):
        cb = cb_ref[:, pl.ds(j * BF, BF)]
        c2 = c2_ref[:, pl.ds(j * BF, BF)]
        dot2 = lax.dot_general(p, cb, (((1,), (0,)), ((), ())),
                               preferred_element_type=jnp.float32)
        d2_ref[:, pl.ds(j * BF, BF)] = (p2 + c2) - dot2

    lane_f = lax.broadcasted_iota(jnp.int32, (1, BF), 1).astype(jnp.float32)
    for rc in range(BN // RC):
        minv = jnp.full((RC, BF), jnp.inf, jnp.float32)
        minjf = jnp.zeros((RC, BF), jnp.float32)
        for j in range(F_BLOCKS):
            d2 = d2_ref[pl.ds(rc * RC, RC), pl.ds(j * BF, BF)]
            better = d2 < minv
            minv = jnp.where(better, d2, minv)
            minjf = jnp.where(better, lane_f + np.float32(j * BF), minjf)
        m = jnp.min(minv, axis=1, keepdims=True)
        cand = jnp.where(minv == m, minjf, np.float32(3e38))
        out_ref[0, 0, pl.ds(rc * RC, RC)] = jnp.min(cand, axis=1).astype(jnp.int32)


def kernel(positions, vertices, faces):
    faces_t = jnp.pad(faces, ((0, F_PAD - N_FACE), (0, 0))).T.reshape(-1)
    vflat = jnp.pad(vertices.reshape(-1), (0, 3))

    ct = _centroids_sc(vflat, faces_t)
    cb = ct[0:3].astype(jnp.bfloat16) * jnp.bfloat16(2.0)
    c2 = ct[3:4]

    grid = N_PAD // BN
    pos_pad = jnp.pad(positions, ((0, N_PAD - N_POS), (0, 0)))
    out = pl.pallas_call(
        _bind_tc_kernel,
        grid=(grid,),
        in_specs=[
            pl.BlockSpec((BN, 3), lambda i: (i, 0)),
            pl.BlockSpec((3, F_PAD), lambda i: (0, 0)),
            pl.BlockSpec((1, F_PAD), lambda i: (0, 0)),
        ],
        out_specs=pl.BlockSpec((1, 1, BN), lambda i: (i, 0, 0)),
        out_shape=jax.ShapeDtypeStruct((grid, 1, BN), jnp.int32),
        scratch_shapes=[pltpu.VMEM((BN, F_BLOCKS * BF), jnp.float32)],
    )(pos_pad, cb, c2)
    return out.reshape(N_PAD)[:N_POS]
